# Initial kernel scaffold; baseline (speedup 1.0000x reference)
import jax
import jax.numpy as jnp
from jax.experimental import pallas as pl


def kernel(node_features, edge_radial, edge_angular, edge_index):
    raise NotImplementedError("write your pallas kernel here")


# speedup vs baseline: 1.5373x; 1.5373x over previous
import functools

import jax
import jax.numpy as jnp
from jax import lax
from jax.experimental import pallas as pl
from jax.experimental.pallas import tpu as pltpu
from jax.experimental.pallas import tpu_sc as plsc

NC = 2
NS = 16
NW = NC * NS

CHUNK = 400


def _mp_kernel(n_nodes, node_dim, rad_dim, ang_dim, n_edges,
               table, src_idx, dst_idx, radial, angular, out,
               sidx_v, didx_v, sbuf, dbuf, rbuf, abuf, sem_s, sem_d):
    per_w = n_edges // NW
    n_chunks = per_w // CHUNK
    wid = lax.axis_index("s") * NC + lax.axis_index("c")
    base_w = wid * per_w

    def body(i, _):
        base = base_w + i * CHUNK
        pltpu.sync_copy(src_idx.at[pl.ds(base, CHUNK)], sidx_v)
        pltpu.sync_copy(dst_idx.at[pl.ds(base, CHUNK)], didx_v)
        cp_s = pltpu.async_copy(table.at[sidx_v], sbuf, sem_s)
        cp_d = pltpu.async_copy(table.at[didx_v], dbuf, sem_d)
        pltpu.sync_copy(radial.at[pl.ds(base, CHUNK), :], rbuf)
        pltpu.sync_copy(angular.at[pl.ds(base, CHUNK), :], abuf)
        cp_s.wait()
        cp_d.wait()
        pltpu.sync_copy(sbuf, out.at[pl.ds(base, CHUNK), pl.ds(0, node_dim)])
        pltpu.sync_copy(dbuf, out.at[pl.ds(base, CHUNK),
                                     pl.ds(node_dim, node_dim)])
        pltpu.sync_copy(rbuf, out.at[pl.ds(base, CHUNK),
                                     pl.ds(2 * node_dim, rad_dim)])
        pltpu.sync_copy(abuf, out.at[pl.ds(base, CHUNK),
                                     pl.ds(2 * node_dim + rad_dim, ang_dim)])
        return 0

    lax.fori_loop(0, n_chunks, body, 0)


def kernel(node_features, edge_radial, edge_angular, edge_index):
    n_nodes, node_dim = node_features.shape
    n_edges, rad_dim = edge_radial.shape
    ang_dim = edge_angular.shape[1]
    out_dim = 2 * node_dim + rad_dim + ang_dim

    src = edge_index[0]
    dst = edge_index[1]

    mesh = plsc.VectorSubcoreMesh(core_axis_name="c", subcore_axis_name="s",
                                  num_cores=NC, num_subcores=NS)
    f = pl.kernel(
        functools.partial(_mp_kernel, n_nodes, node_dim, rad_dim, ang_dim,
                          n_edges),
        out_type=jax.ShapeDtypeStruct((n_edges, out_dim), jnp.float32),
        mesh=mesh,
        scratch_types=[
            pltpu.VMEM((CHUNK,), jnp.int32),
            pltpu.VMEM((CHUNK,), jnp.int32),
            pltpu.VMEM((CHUNK, node_dim), jnp.float32),
            pltpu.VMEM((CHUNK, node_dim), jnp.float32),
            pltpu.VMEM((CHUNK, rad_dim), jnp.float32),
            pltpu.VMEM((CHUNK, ang_dim), jnp.float32),
            pltpu.SemaphoreType.DMA,
            pltpu.SemaphoreType.DMA,
        ],
        compiler_params=pltpu.CompilerParams(use_tc_tiling_on_sc=False),
    )
    return f(node_features, src, dst, edge_radial, edge_angular)


# speedup vs baseline: 1.5867x; 1.0321x over previous
import functools

import jax
import jax.numpy as jnp
from jax import lax
from jax.experimental import pallas as pl
from jax.experimental.pallas import tpu as pltpu
from jax.experimental.pallas import tpu_sc as plsc

NC = 2
NS = 16
NW = NC * NS

CHUNK = 200


def _mp_kernel(n_nodes, node_dim, rad_dim, ang_dim, n_edges,
               table, src_idx, dst_idx, radial, angular, out,
               sidx_a, didx_a, sbuf_a, dbuf_a, rbuf_a, abuf_a,
               sidx_b, didx_b, sbuf_b, dbuf_b, rbuf_b, abuf_b,
               sem_sa, sem_da, sem_ra, sem_wa,
               sem_sb, sem_db, sem_rb, sem_wb):
    per_w = n_edges // NW
    n_pairs = per_w // (2 * CHUNK)
    wid = lax.axis_index("s") * NC + lax.axis_index("c")
    base_w = wid * per_w

    sets = (
        (sidx_a, didx_a, sbuf_a, dbuf_a, rbuf_a, abuf_a,
         sem_sa, sem_da, sem_ra, sem_wa),
        (sidx_b, didx_b, sbuf_b, dbuf_b, rbuf_b, abuf_b,
         sem_sb, sem_db, sem_rb, sem_wb),
    )

    def start(base, s):
        (sidx, didx, sbuf, dbuf, rbuf, abuf, sem_s, sem_d, sem_r, _) = s
        pltpu.sync_copy(src_idx.at[pl.ds(base, CHUNK)], sidx)
        pltpu.sync_copy(dst_idx.at[pl.ds(base, CHUNK)], didx)
        cps = pltpu.async_copy(table.at[sidx], sbuf, sem_s)
        cpd = pltpu.async_copy(table.at[didx], dbuf, sem_d)
        cpr = pltpu.async_copy(radial.at[pl.ds(base, CHUNK), :], rbuf, sem_r)
        cpa = pltpu.async_copy(angular.at[pl.ds(base, CHUNK), :], abuf, sem_r)
        return (cps, cpd, cpr, cpa)

    def write(base, s, cps):
        (sidx, didx, sbuf, dbuf, rbuf, abuf, _, _, _, sem_w) = s
        for cp in cps:
            cp.wait()
        ws = pltpu.async_copy(
            sbuf, out.at[pl.ds(base, CHUNK), pl.ds(0, node_dim)], sem_w)
        wd = pltpu.async_copy(
            dbuf, out.at[pl.ds(base, CHUNK), pl.ds(node_dim, node_dim)],
            sem_w)
        wr = pltpu.async_copy(
            rbuf, out.at[pl.ds(base, CHUNK), pl.ds(2 * node_dim, rad_dim)],
            sem_w)
        wa = pltpu.async_copy(
            abuf,
            out.at[pl.ds(base, CHUNK), pl.ds(2 * node_dim + rad_dim,
                                             ang_dim)],
            sem_w)
        return (ws, wd, wr, wa)

    def body(k, _):
        base0 = base_w + (2 * k) * CHUNK
        base1 = base0 + CHUNK
        cps0 = start(base0, sets[0])
        cps1 = start(base1, sets[1])
        w0 = write(base0, sets[0], cps0)
        w1 = write(base1, sets[1], cps1)
        for cp in w0 + w1:
            cp.wait()
        return 0

    lax.fori_loop(0, n_pairs, body, 0)


def kernel(node_features, edge_radial, edge_angular, edge_index):
    n_nodes, node_dim = node_features.shape
    n_edges, rad_dim = edge_radial.shape
    ang_dim = edge_angular.shape[1]
    out_dim = 2 * node_dim + rad_dim + ang_dim

    src = edge_index[0]
    dst = edge_index[1]

    mesh = plsc.VectorSubcoreMesh(core_axis_name="c", subcore_axis_name="s",
                                  num_cores=NC, num_subcores=NS)
    buf_set = [
        pltpu.VMEM((CHUNK,), jnp.int32),
        pltpu.VMEM((CHUNK,), jnp.int32),
        pltpu.VMEM((CHUNK, node_dim), jnp.float32),
        pltpu.VMEM((CHUNK, node_dim), jnp.float32),
        pltpu.VMEM((CHUNK, rad_dim), jnp.float32),
        pltpu.VMEM((CHUNK, ang_dim), jnp.float32),
    ]
    sem_set = [pltpu.SemaphoreType.DMA] * 4
    f = pl.kernel(
        functools.partial(_mp_kernel, n_nodes, node_dim, rad_dim, ang_dim,
                          n_edges),
        out_type=jax.ShapeDtypeStruct((n_edges, out_dim), jnp.float32),
        mesh=mesh,
        scratch_types=buf_set + buf_set + sem_set + sem_set,
        compiler_params=pltpu.CompilerParams(use_tc_tiling_on_sc=False),
    )
    return f(node_features, src, dst, edge_radial, edge_angular)
